# Initial kernel scaffold; baseline (speedup 1.0000x reference)
import jax
import jax.numpy as jnp
from jax.experimental import pallas as pl


def kernel(x, spec_router_w, spec_router_b, spec_w1, spec_b1, spec_w2, spec_b2, shared_router_w, shared_router_b, sh_w1, sh_b1, sh_w2, sh_b2):
    raise NotImplementedError("write your pallas kernel here")


# speedup vs baseline: 2.5462x; 2.5462x over previous
import functools

import jax
import jax.numpy as jnp
from jax.experimental import pallas as pl
from jax.experimental.pallas import tpu as pltpu

T = 2048
D = 768
H = 3072
E = 8
K = 2
TM = 128
P = T * K
PP = P + E * TM
NT = PP // TM
_SQRT2 = 1.4142135623730951


def _gelu(x):
    return 0.5 * x * (1.0 + jax.lax.erf(x / _SQRT2))


def _router_body(x_ref, srw_ref, srb_ref, shw_ref, shb_ref,
                 idx_ref, p2_ref, meanp_ref, shp_ref):
    x = x_ref[...]
    logits = jnp.dot(x, srw_ref[...], preferred_element_type=jnp.float32) + srb_ref[...]
    m = jnp.max(logits, axis=1, keepdims=True)
    el = jnp.exp(logits - m)
    probs = el / jnp.sum(el, axis=1, keepdims=True)
    iota = jax.lax.broadcasted_iota(jnp.int32, (T, E), 1)
    m1 = jnp.max(probs, axis=1, keepdims=True)
    i1 = jnp.min(jnp.where(probs == m1, iota, E), axis=1, keepdims=True)
    probs_m = jnp.where(iota == i1, -1.0, probs)
    m2 = jnp.max(probs_m, axis=1, keepdims=True)
    i2 = jnp.min(jnp.where(probs_m == m2, iota, E), axis=1, keepdims=True)
    idx_ref[:, 0:1] = i1
    idx_ref[:, 1:2] = i2
    p2_ref[:, 0:1] = m1
    p2_ref[:, 1:2] = m2
    meanp = jnp.sum(probs, axis=0, keepdims=True) / T
    meanp_ref[...] = jnp.concatenate([meanp, jnp.zeros((1, 8), jnp.float32)], axis=1)
    shl = jnp.dot(x, shw_ref[...], preferred_element_type=jnp.float32) + shb_ref[...]
    sm = jnp.max(shl, axis=1, keepdims=True)
    sel = jnp.exp(shl - sm)
    shp_ref[...] = sel / jnp.sum(sel, axis=1, keepdims=True)


def _run_router(xf, srw, srb, shw, shb):
    return pl.pallas_call(
        _router_body,
        out_shape=[
            jax.ShapeDtypeStruct((T, K), jnp.int32),
            jax.ShapeDtypeStruct((T, K), jnp.float32),
            jax.ShapeDtypeStruct((1, 16), jnp.float32),
            jax.ShapeDtypeStruct((T, 2), jnp.float32),
        ],
    )(xf, srw, srb.reshape(1, E), shw, shb.reshape(1, 2))


def _dispatch(idx2, p2, meanp, xf):
    e = idx2.reshape(-1)
    counts = jnp.zeros((E,), jnp.int32).at[e].add(1)
    caps = ((counts + TM - 1) // TM) * TM
    starts = jnp.concatenate([jnp.zeros((1,), jnp.int32), jnp.cumsum(caps)[:-1]])
    order = jnp.argsort(e, stable=True)
    sorted_e = e[order]
    excl = jnp.concatenate([jnp.zeros((1,), jnp.int32), jnp.cumsum(counts)[:-1]])
    within = jnp.arange(P, dtype=jnp.int32) - excl[sorted_e]
    pos_sorted = starts[sorted_e] + within
    pos = jnp.zeros((P,), jnp.int32).at[order].set(pos_sorted)
    tok = jnp.arange(P, dtype=jnp.int32) // K
    x_sorted = jnp.zeros((PP, D), jnp.float32).at[pos].set(xf[tok])
    tile_rows = jnp.arange(NT, dtype=jnp.int32) * TM
    eot = jnp.sum((tile_rows[:, None] >= starts[None, 1:]).astype(jnp.int32), axis=1)
    aux = E * jnp.sum((counts.astype(jnp.float32) / P) * meanp.reshape(-1)[:E])
    return x_sorted, eot, pos, aux


def _ffn_body(eot_ref, x_ref, w1_ref, b1_ref, w2_ref, b2_ref, y_ref):
    h = _gelu(jnp.dot(x_ref[...], w1_ref[0], preferred_element_type=jnp.float32)
              + b1_ref[0])
    y_ref[...] = (jnp.dot(h, w2_ref[0], preferred_element_type=jnp.float32)
                  + b2_ref[0])


def _run_grouped_ffn(eot, x_sorted, w1, b1, w2, b2):
    grid_spec = pltpu.PrefetchScalarGridSpec(
        num_scalar_prefetch=1,
        grid=(NT,),
        in_specs=[
            pl.BlockSpec((TM, D), lambda i, eot: (i, 0)),
            pl.BlockSpec((1, D, H), lambda i, eot: (eot[i], 0, 0)),
            pl.BlockSpec((1, 1, H), lambda i, eot: (eot[i], 0, 0)),
            pl.BlockSpec((1, H, D), lambda i, eot: (eot[i], 0, 0)),
            pl.BlockSpec((1, 1, D), lambda i, eot: (eot[i], 0, 0)),
        ],
        out_specs=pl.BlockSpec((TM, D), lambda i, eot: (i, 0)),
    )
    return pl.pallas_call(
        _ffn_body,
        grid_spec=grid_spec,
        out_shape=jax.ShapeDtypeStruct((PP, D), jnp.float32),
    )(eot, x_sorted, w1, b1.reshape(E, 1, H), w2, b2.reshape(E, 1, D))


def _shared_body(x_ref, w1_ref, b1_ref, w2_ref, b2_ref, shp_ref, out_ref):
    x = x_ref[...]
    acc = jnp.zeros((TM, D), jnp.float32)
    for j in range(2):
        h = _gelu(jnp.dot(x, w1_ref[j], preferred_element_type=jnp.float32)
                  + b1_ref[j:j + 1, :])
        o = jnp.dot(h, w2_ref[j], preferred_element_type=jnp.float32) + b2_ref[j:j + 1, :]
        acc = acc + shp_ref[:, j:j + 1] * o
    out_ref[...] = acc


def _run_shared(xf, w1, b1, w2, b2, shp):
    nt = T // TM
    return pl.pallas_call(
        _shared_body,
        grid=(nt,),
        in_specs=[
            pl.BlockSpec((TM, D), lambda i: (i, 0)),
            pl.BlockSpec((2, D, H), lambda i: (0, 0, 0)),
            pl.BlockSpec((2, H), lambda i: (0, 0)),
            pl.BlockSpec((2, H, D), lambda i: (0, 0, 0)),
            pl.BlockSpec((2, D), lambda i: (0, 0)),
            pl.BlockSpec((TM, 2), lambda i: (i, 0)),
        ],
        out_specs=pl.BlockSpec((TM, D), lambda i: (i, 0)),
        out_shape=jax.ShapeDtypeStruct((T, D), jnp.float32),
    )(xf, w1, b1, w2, b2, shp)


def _combine(y_sorted, pos, p2, shared_out):
    y0 = y_sorted[pos[0::2]]
    y1 = y_sorted[pos[1::2]]
    return shared_out + p2[:, 0:1] * y0 + p2[:, 1:2] * y1


def kernel(x, spec_router_w, spec_router_b, spec_w1, spec_b1, spec_w2, spec_b2,
           shared_router_w, shared_router_b, sh_w1, sh_b1, sh_w2, sh_b2):
    b, s, d = x.shape
    xf = x.reshape(-1, d)
    idx2, p2, meanp, shp = _run_router(xf, spec_router_w, spec_router_b,
                                       shared_router_w, shared_router_b)
    x_sorted, eot, pos, aux = _dispatch(idx2, p2, meanp, xf)
    y_sorted = _run_grouped_ffn(eot, x_sorted, spec_w1, spec_b1, spec_w2, spec_b2)
    shared_out = _run_shared(xf, sh_w1, sh_b1, sh_w2, sh_b2, shp)
    out = _combine(y_sorted, pos, p2, shared_out)
    return out.reshape(b, s, d), aux


# speedup vs baseline: 3.2487x; 1.2759x over previous
import functools

import jax
import jax.numpy as jnp
from jax import lax
from jax.experimental import pallas as pl
from jax.experimental.pallas import tpu as pltpu
from jax.experimental.pallas import tpu_sc as plsc

T = 2048
D = 768
H = 3072
E = 8
K = 2
TM = 128
P = T * K
PP = P + E * TM
NT = PP // TM
_SQRT2 = 1.4142135623730951


def _gelu(x):
    return 0.5 * x * (1.0 + jax.lax.erf(x / _SQRT2))


def _router_body(x_ref, srw_ref, srb_ref, shw_ref, shb_ref,
                 idx_ref, p2_ref, meanp_ref, shp_ref):
    x = x_ref[...]
    logits = jnp.dot(x, srw_ref[...], preferred_element_type=jnp.float32) + srb_ref[...]
    m = jnp.max(logits, axis=1, keepdims=True)
    el = jnp.exp(logits - m)
    probs = el / jnp.sum(el, axis=1, keepdims=True)
    iota = jax.lax.broadcasted_iota(jnp.int32, (T, E), 1)
    m1 = jnp.max(probs, axis=1, keepdims=True)
    i1 = jnp.min(jnp.where(probs == m1, iota, E), axis=1, keepdims=True)
    probs_m = jnp.where(iota == i1, -1.0, probs)
    m2 = jnp.max(probs_m, axis=1, keepdims=True)
    i2 = jnp.min(jnp.where(probs_m == m2, iota, E), axis=1, keepdims=True)
    idx_ref[:, 0:1] = i1
    idx_ref[:, 1:2] = i2
    p2_ref[:, 0:1] = m1
    p2_ref[:, 1:2] = m2
    meanp = jnp.sum(probs, axis=0, keepdims=True) / T
    meanp_ref[...] = jnp.concatenate([meanp, jnp.zeros((1, 8), jnp.float32)], axis=1)
    shl = jnp.dot(x, shw_ref[...], preferred_element_type=jnp.float32) + shb_ref[...]
    sm = jnp.max(shl, axis=1, keepdims=True)
    sel = jnp.exp(shl - sm)
    shp_ref[...] = sel / jnp.sum(sel, axis=1, keepdims=True)


def _run_router(xf, srw, srb, shw, shb):
    return pl.pallas_call(
        _router_body,
        out_shape=[
            jax.ShapeDtypeStruct((T, K), jnp.int32),
            jax.ShapeDtypeStruct((T, K), jnp.float32),
            jax.ShapeDtypeStruct((1, 16), jnp.float32),
            jax.ShapeDtypeStruct((T, 2), jnp.float32),
        ],
    )(xf, srw, srb.reshape(1, E), shw, shb.reshape(1, 2))


NW = 32
CHUNK = P // NW
NCH = CHUNK // 16
NCHALL = P // 16


def _prefix_sum16(v):
    iot = lax.iota(jnp.int32, 16)
    for s in (1, 2, 4, 8):
        g = v.at[jnp.maximum(iot - s, 0)].get(mode="promise_in_bounds")
        v = v + jnp.where(iot >= s, g, jnp.zeros_like(v))
    return v


def _sc_dispatch_body(idx_hbm, meanp_hbm, xf_hbm,
                      xs_out, eot_out, pos_out, aux_out,
                      idx_v, pos_v, tok_v, xrow_v, eot_v, meanp_v, aux_v,
                      sem1, sem2):
    cid = lax.axis_index("c")
    sid = lax.axis_index("s")
    wid = sid * 2 + cid
    pltpu.sync_copy(idx_hbm, idx_v)
    my_first = wid * NCH

    def hist(lo, hi):
        def body(c, carry):
            e = idx_v[pl.ds(c * 16, 16)]
            return tuple(carry[j] + plsc.all_reduce_population_count(e == j)
                         for j in range(E))
        init = tuple(jnp.zeros((16,), jnp.int32) for _ in range(E))
        return lax.fori_loop(lo, hi, body, init)

    cb = hist(0, my_first)
    rest = hist(my_first, NCHALL)
    tot = [cb[j] + rest[j] for j in range(E)]
    starts = [jnp.zeros((16,), jnp.int32)] * E
    for j in range(1, E):
        cap = ((tot[j - 1] + (TM - 1)) >> 7) << 7
        starts[j] = starts[j - 1] + cap

    for v in range(3):
        row = (lax.iota(jnp.int32, 16) + v * 16) * TM
        ev = jnp.zeros((16,), jnp.int32)
        for j in range(1, E):
            ev = jnp.where(row >= starts[j], j, ev)
        eot_v[pl.ds(v * 16, 16)] = ev

    off = [starts[j] + cb[j] for j in range(E)]
    base_pair = wid * CHUNK
    for c in range(NCH):
        e = idx_v[pl.ds((my_first + c) * 16, 16)]
        pvec = jnp.zeros((16,), jnp.int32)
        for j in range(E):
            m = e == j
            r = _prefix_sum16(jnp.where(m, 1, 0))
            pvec = jnp.where(m, off[j] + r - 1, pvec)
            off[j] = off[j] + plsc.all_reduce_population_count(m)
        pos_v[pl.ds(c * 16, 16)] = pvec
        tok_v[pl.ds(c * 16, 16)] = (base_pair + c * 16
                                    + lax.iota(jnp.int32, 16)) >> 1
    pltpu.sync_copy(pos_v, pos_out.at[pl.ds(base_pair, CHUNK)])

    pltpu.async_copy(xf_hbm.at[tok_v], xrow_v, sem1).wait()
    pltpu.async_copy(xrow_v, xs_out.at[pos_v], sem2).wait()

    @pl.when(wid == 0)
    def _():
        pltpu.sync_copy(eot_v.at[pl.ds(0, NT)], eot_out)
        pltpu.sync_copy(meanp_hbm, meanp_v)
        tv = jnp.zeros((16,), jnp.float32)
        for j in range(E):
            tv = jnp.where(lax.iota(jnp.int32, 16) == j,
                           tot[j].astype(jnp.float32), tv)
        av = _prefix_sum16(meanp_v[...] * tv)
        aux = av.at[jnp.full((16,), 15, jnp.int32)].get(
            mode="promise_in_bounds") * (float(E) / float(P))
        aux_v[...] = aux
        pltpu.sync_copy(aux_v.at[pl.ds(0, 8)], aux_out)


def _dispatch(idx2, meanp, xf):
    mesh = plsc.VectorSubcoreMesh(core_axis_name="c", subcore_axis_name="s")
    f = pl.kernel(
        _sc_dispatch_body, mesh=mesh,
        out_type=[
            jax.ShapeDtypeStruct((PP, D), jnp.float32),
            jax.ShapeDtypeStruct((NT,), jnp.int32),
            jax.ShapeDtypeStruct((P,), jnp.int32),
            jax.ShapeDtypeStruct((8,), jnp.float32),
        ],
        scratch_types=[
            pltpu.VMEM((P,), jnp.int32),
            pltpu.VMEM((CHUNK,), jnp.int32),
            pltpu.VMEM((CHUNK,), jnp.int32),
            pltpu.VMEM((CHUNK, D), jnp.float32),
            pltpu.VMEM((48,), jnp.int32),
            pltpu.VMEM((16,), jnp.float32),
            pltpu.VMEM((16,), jnp.float32),
            pltpu.SemaphoreType.DMA,
            pltpu.SemaphoreType.DMA,
        ],
        compiler_params=pltpu.CompilerParams(needs_layout_passes=False),
    )
    x_sorted, eot, pos, aux8 = f(idx2.reshape(-1), meanp.reshape(-1), xf)
    return x_sorted, eot, pos, aux8[0]


def _ffn_body(eot_ref, x_ref, w1_ref, b1_ref, w2_ref, b2_ref, y_ref):
    h = _gelu(jnp.dot(x_ref[...], w1_ref[0], preferred_element_type=jnp.float32)
              + b1_ref[0])
    y_ref[...] = (jnp.dot(h, w2_ref[0], preferred_element_type=jnp.float32)
                  + b2_ref[0])


def _run_grouped_ffn(eot, x_sorted, w1, b1, w2, b2):
    grid_spec = pltpu.PrefetchScalarGridSpec(
        num_scalar_prefetch=1,
        grid=(NT,),
        in_specs=[
            pl.BlockSpec((TM, D), lambda i, eot: (i, 0)),
            pl.BlockSpec((1, D, H), lambda i, eot: (eot[i], 0, 0)),
            pl.BlockSpec((1, 1, H), lambda i, eot: (eot[i], 0, 0)),
            pl.BlockSpec((1, H, D), lambda i, eot: (eot[i], 0, 0)),
            pl.BlockSpec((1, 1, D), lambda i, eot: (eot[i], 0, 0)),
        ],
        out_specs=pl.BlockSpec((TM, D), lambda i, eot: (i, 0)),
    )
    return pl.pallas_call(
        _ffn_body,
        grid_spec=grid_spec,
        out_shape=jax.ShapeDtypeStruct((PP, D), jnp.float32),
    )(eot, x_sorted, w1, b1.reshape(E, 1, H), w2, b2.reshape(E, 1, D))


def _shared_body(x_ref, w1_ref, b1_ref, w2_ref, b2_ref, shp_ref, out_ref):
    x = x_ref[...]
    acc = jnp.zeros((TM, D), jnp.float32)
    for j in range(2):
        h = _gelu(jnp.dot(x, w1_ref[j], preferred_element_type=jnp.float32)
                  + b1_ref[j:j + 1, :])
        o = jnp.dot(h, w2_ref[j], preferred_element_type=jnp.float32) + b2_ref[j:j + 1, :]
        acc = acc + shp_ref[:, j:j + 1] * o
    out_ref[...] = acc


def _run_shared(xf, w1, b1, w2, b2, shp):
    nt = T // TM
    return pl.pallas_call(
        _shared_body,
        grid=(nt,),
        in_specs=[
            pl.BlockSpec((TM, D), lambda i: (i, 0)),
            pl.BlockSpec((2, D, H), lambda i: (0, 0, 0)),
            pl.BlockSpec((2, H), lambda i: (0, 0)),
            pl.BlockSpec((2, H, D), lambda i: (0, 0, 0)),
            pl.BlockSpec((2, D), lambda i: (0, 0)),
            pl.BlockSpec((TM, 2), lambda i: (i, 0)),
        ],
        out_specs=pl.BlockSpec((TM, D), lambda i: (i, 0)),
        out_shape=jax.ShapeDtypeStruct((T, D), jnp.float32),
    )(xf, w1, b1, w2, b2, shp)


TOKW = T // NW
THALF = TOKW // 2
NV = D // 16


def _sc_combine_body(y_hbm, pos_hbm, pw_hbm, sh_hbm, out_hbm,
                     pos_v, pw_v, yrow_v, acc_v, sem):
    cid = lax.axis_index("c")
    sid = lax.axis_index("s")
    wid = sid * 2 + cid
    for half in range(2):
        tbase = wid * TOKW + half * THALF
        pbase = tbase * 2
        pltpu.sync_copy(pos_hbm.at[pl.ds(pbase, 2 * THALF)], pos_v)
        pltpu.sync_copy(pw_hbm.at[pl.ds(pbase, 2 * THALF)], pw_v)
        pltpu.async_copy(y_hbm.at[pos_v], yrow_v, sem).wait()
        pltpu.sync_copy(sh_hbm.at[pl.ds(tbase, THALF)], acc_v)

        def tok_body(i, _):
            w0 = plsc.load_gather(pw_v, [jnp.broadcast_to(2 * i, (16,))])
            w1 = plsc.load_gather(pw_v, [jnp.broadcast_to(2 * i + 1, (16,))])
            for c in range(NV):
                y0 = yrow_v[2 * i, pl.ds(c * 16, 16)]
                y1 = yrow_v[2 * i + 1, pl.ds(c * 16, 16)]
                a = acc_v[i, pl.ds(c * 16, 16)]
                acc_v[i, pl.ds(c * 16, 16)] = a + w0 * y0 + w1 * y1
            return 0

        lax.fori_loop(0, THALF, tok_body, 0)
        pltpu.sync_copy(acc_v, out_hbm.at[pl.ds(tbase, THALF)])


def _combine(y_sorted, pos, p2, shared_out):
    mesh = plsc.VectorSubcoreMesh(core_axis_name="c", subcore_axis_name="s")
    f = pl.kernel(
        _sc_combine_body, mesh=mesh,
        out_type=jax.ShapeDtypeStruct((T, D), jnp.float32),
        scratch_types=[
            pltpu.VMEM((2 * THALF,), jnp.int32),
            pltpu.VMEM((2 * THALF,), jnp.float32),
            pltpu.VMEM((2 * THALF, D), jnp.float32),
            pltpu.VMEM((THALF, D), jnp.float32),
            pltpu.SemaphoreType.DMA,
        ],
        compiler_params=pltpu.CompilerParams(needs_layout_passes=False),
    )
    return f(y_sorted, pos, p2.reshape(-1), shared_out)


def kernel(x, spec_router_w, spec_router_b, spec_w1, spec_b1, spec_w2, spec_b2,
           shared_router_w, shared_router_b, sh_w1, sh_b1, sh_w2, sh_b2):
    b, s, d = x.shape
    xf = x.reshape(-1, d)
    idx2, p2, meanp, shp = _run_router(xf, spec_router_w, spec_router_b,
                                       shared_router_w, shared_router_b)
    x_sorted, eot, pos, aux = _dispatch(idx2, meanp, xf)
    y_sorted = _run_grouped_ffn(eot, x_sorted, spec_w1, spec_b1, spec_w2, spec_b2)
    shared_out = _run_shared(xf, sh_w1, sh_b1, sh_w2, sh_b2, shp)
    out = _combine(y_sorted, pos, p2, shared_out)
    return out.reshape(b, s, d), aux
